# baseline (speedup 1.0000x reference)
import functools

import jax
import jax.numpy as jnp
from jax import lax
from jax.experimental import pallas as pl
from jax.experimental.pallas import tpu as pltpu
from jax.experimental.pallas import tpu_sc as plsc


_CHUNK = 3200


def _vocab_f_body(te, pe, tw, tb, pw1, pb1, pw2, pb2, out,
                  twb, pw1b, pw2b):
    @pl.when(pl.program_id(0) == 0)
    def _cast_weights():
        twb[...] = tw[...].astype(jnp.bfloat16)
        pw1b[...] = pw1[...].astype(jnp.bfloat16)
        pw2b[...] = pw2[...].astype(jnp.bfloat16)

    teb = te[...].astype(jnp.bfloat16)
    peb = pe[...].astype(jnp.bfloat16)
    tf = jnp.dot(teb, twb[...], preferred_element_type=jnp.float32) + tb[...]
    h = jnp.dot(peb, pw1b[...], preferred_element_type=jnp.float32) + pb1[...]
    hr = jnp.maximum(h, 0.0).astype(jnp.bfloat16)
    pf = jnp.dot(hr, pw2b[...], preferred_element_type=jnp.float32) + pb2[...]
    d = pf - tf
    out[...] = jnp.mean(d * d, axis=1, keepdims=True)


def _vocab_f(t_emb, t_W, t_b2d, p_emb, p_W1, p_b1_2d, p_W2, p_b2_2d):
    V, H = t_emb.shape
    O = t_W.shape[1]
    grid = (V + _CHUNK - 1) // _CHUNK
    return pl.pallas_call(
        _vocab_f_body,
        grid=(grid,),
        in_specs=[
            pl.BlockSpec((_CHUNK, H), lambda i: (i, 0)),
            pl.BlockSpec((_CHUNK, H), lambda i: (i, 0)),
            pl.BlockSpec((H, O), lambda i: (0, 0)),
            pl.BlockSpec((1, O), lambda i: (0, 0)),
            pl.BlockSpec((H, O), lambda i: (0, 0)),
            pl.BlockSpec((1, O), lambda i: (0, 0)),
            pl.BlockSpec((O, O), lambda i: (0, 0)),
            pl.BlockSpec((1, O), lambda i: (0, 0)),
        ],
        out_specs=pl.BlockSpec((_CHUNK, 1), lambda i: (i, 0)),
        out_shape=jax.ShapeDtypeStruct((V, 1), jnp.float32),
        scratch_shapes=[
            pltpu.VMEM((H, O), jnp.bfloat16),
            pltpu.VMEM((H, O), jnp.bfloat16),
            pltpu.VMEM((O, O), jnp.bfloat16),
        ],
    )(t_emb, p_emb, t_W, t_b2d, p_W1, p_b1_2d, p_W2, p_b2_2d)


def _sc_reward(f_flat, ids_flat, B, L):
    V = f_flat.shape[0]
    info = plsc.get_sparse_core_info()
    NW = info.num_cores * info.num_subcores
    LANES = info.num_lanes
    rows_per_w = B // NW
    ids_per_w = rows_per_w * L
    row_blocks = rows_per_w // LANES
    inv_l = 1.0 / float(L)

    mesh = plsc.VectorSubcoreMesh(core_axis_name="c", subcore_axis_name="s")

    @functools.partial(
        pl.kernel,
        mesh=mesh,
        out_type=jax.ShapeDtypeStruct((B,), jnp.float32),
        scratch_types=[
            pltpu.VMEM((V,), jnp.float32),
            pltpu.VMEM((ids_per_w,), jnp.int32),
            pltpu.VMEM((rows_per_w,), jnp.float32),
        ],
        compiler_params=pltpu.CompilerParams(needs_layout_passes=False),
    )
    def k(f_hbm, ids_hbm, out_hbm, f_v, ids_v, rew_v):
        wid = lax.axis_index("s") * info.num_cores + lax.axis_index("c")
        base = wid * ids_per_w
        pltpu.sync_copy(f_hbm, f_v)
        pltpu.sync_copy(ids_hbm.at[pl.ds(base, ids_per_w)], ids_v)
        iota = lax.iota(jnp.int32, LANES)
        for rb in range(row_blocks):
            rows = rb * LANES + iota
            acc = jnp.zeros((LANES,), jnp.float32)
            for j in range(L):
                pos = rows * L + j
                tok = plsc.load_gather(ids_v, [pos])
                acc = acc + plsc.load_gather(f_v, [tok])
            rew_v[pl.ds(rb * LANES, LANES)] = acc * inv_l
        pltpu.sync_copy(rew_v, out_hbm.at[pl.ds(wid * rows_per_w, rows_per_w)])

    return k(f_flat, ids_flat)


def kernel(token_ids, t_emb, t_W, t_b, p_emb, p_W1, p_b1, p_W2, p_b2):
    B, L = token_ids.shape
    f = _vocab_f(
        t_emb, t_W, t_b.reshape(1, -1),
        p_emb, p_W1, p_b1.reshape(1, -1), p_W2, p_b2.reshape(1, -1),
    )
    f_flat = f.reshape(-1)
    ids_flat = token_ids.reshape(-1).astype(jnp.int32)
    reward = _sc_reward(f_flat, ids_flat, B, L)
    return (jnp.mean(reward), reward)


# speedup vs baseline: 1.0084x; 1.0084x over previous
import functools

import jax
import jax.numpy as jnp
from jax import lax
from jax.experimental import pallas as pl
from jax.experimental.pallas import tpu as pltpu
from jax.experimental.pallas import tpu_sc as plsc


_CHUNK = 3072


def _vocab_f_body(te, pe, tw, tb, pw1, pb1, pw2, pb2, out,
                  twb, pw1b, pw2b):
    @pl.when(pl.program_id(0) == 0)
    def _cast_weights():
        twb[...] = tw[...].astype(jnp.bfloat16)
        pw1b[...] = pw1[...].astype(jnp.bfloat16)
        pw2b[...] = pw2[...].astype(jnp.bfloat16)

    teb = te[...].astype(jnp.bfloat16)
    peb = pe[...].astype(jnp.bfloat16)
    tf = jnp.dot(teb, twb[...], preferred_element_type=jnp.float32) + tb[...]
    h = jnp.dot(peb, pw1b[...], preferred_element_type=jnp.float32) + pb1[...]
    hr = jnp.maximum(h, 0.0).astype(jnp.bfloat16)
    pf = jnp.dot(hr, pw2b[...], preferred_element_type=jnp.float32) + pb2[...]
    d = pf - tf
    out[...] = jnp.mean(d * d, axis=1, keepdims=True)


def _vocab_f(t_emb, t_W, t_b2d, p_emb, p_W1, p_b1_2d, p_W2, p_b2_2d):
    V, H = t_emb.shape
    O = t_W.shape[1]
    grid = (V + _CHUNK - 1) // _CHUNK
    return pl.pallas_call(
        _vocab_f_body,
        grid=(grid,),
        in_specs=[
            pl.BlockSpec((_CHUNK, H), lambda i: (i, 0)),
            pl.BlockSpec((_CHUNK, H), lambda i: (i, 0)),
            pl.BlockSpec((H, O), lambda i: (0, 0)),
            pl.BlockSpec((1, O), lambda i: (0, 0)),
            pl.BlockSpec((H, O), lambda i: (0, 0)),
            pl.BlockSpec((1, O), lambda i: (0, 0)),
            pl.BlockSpec((O, O), lambda i: (0, 0)),
            pl.BlockSpec((1, O), lambda i: (0, 0)),
        ],
        out_specs=pl.BlockSpec((_CHUNK, 1), lambda i: (i, 0)),
        out_shape=jax.ShapeDtypeStruct((V, 1), jnp.float32),
        scratch_shapes=[
            pltpu.VMEM((H, O), jnp.bfloat16),
            pltpu.VMEM((H, O), jnp.bfloat16),
            pltpu.VMEM((O, O), jnp.bfloat16),
        ],
    )(t_emb, p_emb, t_W, t_b2d, p_W1, p_b1_2d, p_W2, p_b2_2d)


def _sc_reward(f_flat, ids_2d, B, L):
    V = f_flat.shape[0]
    info = plsc.get_sparse_core_info()
    NW = info.num_cores * info.num_subcores
    LANES = info.num_lanes
    rows_per_w = B // NW
    ids_per_w = rows_per_w * L
    row_blocks = rows_per_w // LANES
    inv_l = 1.0 / float(L)

    mesh = plsc.VectorSubcoreMesh(core_axis_name="c", subcore_axis_name="s")

    @functools.partial(
        pl.kernel,
        mesh=mesh,
        out_type=jax.ShapeDtypeStruct((B,), jnp.float32),
        scratch_types=[
            pltpu.VMEM((V,), jnp.float32),
            pltpu.VMEM((rows_per_w, L), jnp.int32),
            pltpu.VMEM((rows_per_w,), jnp.float32),
        ],
        compiler_params=pltpu.CompilerParams(needs_layout_passes=False),
    )
    def k(f_hbm, ids_hbm, out_hbm, f_v, ids_v, rew_v):
        wid = lax.axis_index("s") * info.num_cores + lax.axis_index("c")
        pltpu.sync_copy(f_hbm, f_v)
        pltpu.sync_copy(ids_hbm.at[pl.ds(wid * rows_per_w, rows_per_w), :], ids_v)
        iota = lax.iota(jnp.int32, LANES)
        for rb in range(row_blocks):
            rows = rb * LANES + iota
            acc = jnp.zeros((LANES,), jnp.float32)
            for j in range(L):
                jvec = jnp.full((LANES,), j, jnp.int32)
                tok = plsc.load_gather(ids_v, [rows, jvec])
                acc = acc + plsc.load_gather(f_v, [tok])
            rew_v[pl.ds(rb * LANES, LANES)] = acc * inv_l
        pltpu.sync_copy(rew_v, out_hbm.at[pl.ds(wid * rows_per_w, rows_per_w)])

    return k(f_flat, ids_2d)


def kernel(token_ids, t_emb, t_W, t_b, p_emb, p_W1, p_b1, p_W2, p_b2):
    B, L = token_ids.shape
    f = _vocab_f(
        t_emb, t_W, t_b.reshape(1, -1),
        p_emb, p_W1, p_b1.reshape(1, -1), p_W2, p_b2.reshape(1, -1),
    )
    f_flat = f.reshape(-1)
    reward = _sc_reward(f_flat, token_ids.astype(jnp.int32), B, L)
    return (jnp.mean(reward), reward)
